# Initial kernel scaffold; baseline (speedup 1.0000x reference)
import jax
import jax.numpy as jnp
from jax.experimental import pallas as pl


def kernel(z, embed_weight):
    raise NotImplementedError("write your pallas kernel here")


# speedup vs baseline: 6.9949x; 6.9949x over previous
import functools

import jax
import jax.numpy as jnp
from jax import lax
from jax.experimental import pallas as pl
from jax.experimental.pallas import tpu as pltpu
from jax.experimental.pallas import tpu_sc as plsc

N_E = 8192
E_DIM = 256
BETA = 0.25
B_SZ = 16
HW = 1024
N_TOK = 16384
BM = 512
A_GRID = N_TOK // BM
TB = 2048
C_GRID = N_TOK // TB

SC_NC = 2
SC_NS = 16
SC_NW = SC_NC * SC_NS
PER_W = N_TOK // SC_NW
CHUNK = 128
N_CH = PER_W // CHUNK


ARG_BOUNDS = ((0, 2736), (2736, 5472), (5472, N_E))


def _assign_body(z_ref, emb_ref, zz_ref, ee_ref, idx_ref, cnt_ref):
    g = pl.program_id(0)
    zb = z_ref[...].reshape(E_DIM, BM)
    emb = emb_ref[...]
    mm = lax.dot_general(emb.astype(jnp.bfloat16), zb.astype(jnp.bfloat16),
                         (((1,), (0,)), ((), ())),
                         preferred_element_type=jnp.float32)
    ee = ee_ref[...]
    zz = zz_ref[...].reshape(BM)
    s = 2.0 * mm - (zz[None, :] + ee)
    row = lax.broadcasted_iota(jnp.int32, (N_E, 1), 0)
    acc_v = None
    acc_i = None
    for c, (lo, hi) in enumerate(ARG_BOUNDS):
        mask = (row >= lo) & (row < hi)
        sc = jnp.where(mask, s, -jnp.inf)
        i_c = jnp.argmax(sc, axis=0)
        v_c = jnp.max(sc, axis=0)
        if c == 0:
            acc_i = i_c
            acc_v = v_c
        else:
            take = v_c > acc_v
            acc_i = jnp.where(take, i_c, acc_i)
            acc_v = jnp.where(take, v_c, acc_v)
        acc_v = acc_v.astype(jnp.bfloat16).astype(jnp.float32)
    idx = acc_i
    idx_ref[0, 0, :] = idx

    @pl.when(g == 0)
    def _():
        cnt_ref[...] = jnp.zeros_like(cnt_ref)

    onehot = (idx[None, :] ==
              lax.broadcasted_iota(jnp.int32, (N_E, 1), 0)).astype(jnp.float32)
    cnt_ref[...] += jnp.sum(onehot, axis=1, keepdims=True)


def _finish_body(g_ref, z_ref, cnt_ref, loss_ref, perp_ref, acc_ref):
    g = pl.program_id(0)
    diff = g_ref[...] - z_ref[...]
    part = jnp.sum(diff * diff)

    @pl.when(g == 0)
    def _():
        acc_ref[0] = part

    @pl.when(g > 0)
    def _():
        acc_ref[0] += part

    @pl.when(g == C_GRID - 1)
    def _():
        mean_sq = acc_ref[0] * (1.0 / (N_TOK * E_DIM))
        loss_ref[...] = jnp.full((1, 1), (1.0 + BETA) * mean_sq, jnp.float32)
        e_mean = cnt_ref[...] * (1.0 / N_TOK)
        ent = jnp.sum(e_mean * jnp.log(e_mean + 1e-10))
        perp_ref[...] = jnp.full((1, 1), jnp.exp(-ent), jnp.float32)


_assign_call = pl.pallas_call(
    _assign_body,
    grid=(A_GRID,),
    in_specs=[
        pl.BlockSpec((1, E_DIM, BM), lambda g: (g // (HW // BM), 0, g % (HW // BM))),
        pl.BlockSpec((N_E, E_DIM), lambda g: (0, 0)),
        pl.BlockSpec((1, 1, BM), lambda g: (g, 0, 0)),
        pl.BlockSpec((N_E, 1), lambda g: (0, 0)),
    ],
    out_specs=[
        pl.BlockSpec((1, 1, BM), lambda g: (g, 0, 0)),
        pl.BlockSpec((N_E, 1), lambda g: (0, 0)),
    ],
    out_shape=[
        jax.ShapeDtypeStruct((A_GRID, 1, BM), jnp.int32),
        jax.ShapeDtypeStruct((N_E, 1), jnp.float32),
    ],
)

_finish_call = pl.pallas_call(
    _finish_body,
    grid=(C_GRID,),
    in_specs=[
        pl.BlockSpec((TB, E_DIM), lambda g: (g, 0)),
        pl.BlockSpec((TB, E_DIM), lambda g: (g, 0)),
        pl.BlockSpec((N_E, 1), lambda g: (0, 0)),
    ],
    out_specs=[
        pl.BlockSpec((1, 1), lambda g: (0, 0)),
        pl.BlockSpec((1, 1), lambda g: (0, 0)),
    ],
    out_shape=[
        jax.ShapeDtypeStruct((1, 1), jnp.float32),
        jax.ShapeDtypeStruct((1, 1), jnp.float32),
    ],
    scratch_shapes=[pltpu.SMEM((1,), jnp.float32)],
)


def _sc_gather_body(table_hbm, idx_hbm, out_hbm, idx_v, rows0, rows1, sem0, sem1):
    wid = lax.axis_index("s") * SC_NC + lax.axis_index("c")
    base = wid * PER_W
    pltpu.sync_copy(idx_hbm.at[pl.ds(wid * N_CH, N_CH)], idx_v)
    bufs = (rows0, rows1)
    sems = (sem0, sem1)
    cps = [None] * N_CH
    cps[0] = pltpu.async_copy(table_hbm.at[idx_v.at[0]], bufs[0], sems[0])
    for c in range(N_CH):
        if c + 1 < N_CH:
            cps[c + 1] = pltpu.async_copy(
                table_hbm.at[idx_v.at[c + 1]], bufs[(c + 1) % 2], sems[(c + 1) % 2])
        cps[c].wait()
        pltpu.sync_copy(bufs[c % 2], out_hbm.at[pl.ds(base + c * CHUNK, CHUNK)])


@functools.cache
def _sc_gather_call():
    return pl.kernel(
        _sc_gather_body,
        mesh=plsc.VectorSubcoreMesh(
            core_axis_name="c", subcore_axis_name="s",
            num_cores=SC_NC, num_subcores=SC_NS),
        out_type=jax.ShapeDtypeStruct((N_TOK, E_DIM), jnp.float32),
        scratch_types=[
            pltpu.VMEM((N_CH, CHUNK), jnp.int32),
            pltpu.VMEM((CHUNK, E_DIM), jnp.float32),
            pltpu.VMEM((CHUNK, E_DIM), jnp.float32),
            pltpu.SemaphoreType.DMA,
            pltpu.SemaphoreType.DMA,
        ],
    )


def kernel(z, embed_weight):
    z3 = z.reshape(B_SZ, E_DIM, HW)
    zf = jnp.transpose(z, (0, 2, 3, 1)).reshape(-1, E_DIM)
    zz = jnp.sum(zf ** 2, axis=1)
    ee = jnp.sum(embed_weight ** 2, axis=1)
    idx_blocks, counts = _assign_call(
        z3, embed_weight, zz.reshape(A_GRID, 1, BM), ee.reshape(N_E, 1))
    idx2d = idx_blocks.reshape(SC_NW * N_CH, CHUNK)
    z_q = _sc_gather_call()(embed_weight, idx2d)
    loss2, perp2 = _finish_call(z_q, z.reshape(N_TOK, E_DIM), counts)
    return (z_q.reshape(z.shape), loss2[0, 0], perp2[0, 0],
            idx_blocks.reshape(B_SZ, 32, 32))


# speedup vs baseline: 8.0370x; 1.1490x over previous
import functools

import jax
import jax.numpy as jnp
from jax import lax
from jax.experimental import pallas as pl
from jax.experimental.pallas import tpu as pltpu
from jax.experimental.pallas import tpu_sc as plsc

N_E = 8192
E_DIM = 256
BETA = 0.25
B_SZ = 16
HW = 1024
N_TOK = 16384
BM = 512
A_GRID = N_TOK // BM
TB = 2048
C_GRID = N_TOK // TB

SC_NC = 2
SC_NS = 16
SC_NW = SC_NC * SC_NS
PER_W = N_TOK // SC_NW
CHUNK = 128
N_CH = PER_W // CHUNK


ARG_BOUNDS = ((0, 2736), (2736, 5472), (5472, N_E))


def _assign_body(z_ref, emb_ref, zz_ref, ee_ref, idx_ref, cnt_ref):
    g = pl.program_id(0)
    zb = z_ref[...].reshape(E_DIM, BM)
    emb = emb_ref[...]
    mm = lax.dot_general(emb.astype(jnp.bfloat16), zb.astype(jnp.bfloat16),
                         (((1,), (0,)), ((), ())),
                         preferred_element_type=jnp.float32)
    ee = ee_ref[...]
    zz = zz_ref[...].reshape(BM)
    s = 2.0 * mm - (zz[None, :] + ee)
    acc_v = None
    acc_i = None
    for c, (lo, hi) in enumerate(ARG_BOUNDS):
        sc = s[lo:hi, :]
        i_c = jnp.argmax(sc, axis=0) + lo
        v_c = jnp.max(sc, axis=0)
        if c == 0:
            acc_i = i_c
            acc_v = v_c
        else:
            take = v_c > acc_v
            acc_i = jnp.where(take, i_c, acc_i)
            acc_v = jnp.where(take, v_c, acc_v)
        acc_v = acc_v.astype(jnp.bfloat16).astype(jnp.float32)
    idx = acc_i
    idx_ref[0, 0, :] = idx

    @pl.when(g == 0)
    def _():
        cnt_ref[...] = jnp.zeros_like(cnt_ref)

    onehot = (idx[None, :] ==
              lax.broadcasted_iota(jnp.int32, (N_E, 1), 0)).astype(jnp.float32)
    cnt_ref[...] += jnp.sum(onehot, axis=1, keepdims=True)


def _finish_body(g_ref, z_ref, cnt_ref, loss_ref, perp_ref, acc_ref):
    g = pl.program_id(0)
    diff = g_ref[...] - z_ref[...]
    part = jnp.sum(diff * diff)

    @pl.when(g == 0)
    def _():
        acc_ref[0] = part

    @pl.when(g > 0)
    def _():
        acc_ref[0] += part

    @pl.when(g == C_GRID - 1)
    def _():
        mean_sq = acc_ref[0] * (1.0 / (N_TOK * E_DIM))
        loss_ref[...] = jnp.full((1, 1), (1.0 + BETA) * mean_sq, jnp.float32)
        e_mean = cnt_ref[...] * (1.0 / N_TOK)
        ent = jnp.sum(e_mean * jnp.log(e_mean + 1e-10))
        perp_ref[...] = jnp.full((1, 1), jnp.exp(-ent), jnp.float32)


_assign_call = pl.pallas_call(
    _assign_body,
    grid=(A_GRID,),
    in_specs=[
        pl.BlockSpec((1, E_DIM, BM), lambda g: (g // (HW // BM), 0, g % (HW // BM))),
        pl.BlockSpec((N_E, E_DIM), lambda g: (0, 0)),
        pl.BlockSpec((1, 1, BM), lambda g: (g, 0, 0)),
        pl.BlockSpec((N_E, 1), lambda g: (0, 0)),
    ],
    out_specs=[
        pl.BlockSpec((1, 1, BM), lambda g: (g, 0, 0)),
        pl.BlockSpec((N_E, 1), lambda g: (0, 0)),
    ],
    out_shape=[
        jax.ShapeDtypeStruct((A_GRID, 1, BM), jnp.int32),
        jax.ShapeDtypeStruct((N_E, 1), jnp.float32),
    ],
)

_finish_call = pl.pallas_call(
    _finish_body,
    grid=(C_GRID,),
    in_specs=[
        pl.BlockSpec((TB, E_DIM), lambda g: (g, 0)),
        pl.BlockSpec((TB, E_DIM), lambda g: (g, 0)),
        pl.BlockSpec((N_E, 1), lambda g: (0, 0)),
    ],
    out_specs=[
        pl.BlockSpec((1, 1), lambda g: (0, 0)),
        pl.BlockSpec((1, 1), lambda g: (0, 0)),
    ],
    out_shape=[
        jax.ShapeDtypeStruct((1, 1), jnp.float32),
        jax.ShapeDtypeStruct((1, 1), jnp.float32),
    ],
    scratch_shapes=[pltpu.SMEM((1,), jnp.float32)],
)


def _sc_gather_body(table_hbm, idx_hbm, out_hbm, idx_v, rows0, rows1, sem0, sem1):
    wid = lax.axis_index("s") * SC_NC + lax.axis_index("c")
    base = wid * PER_W
    pltpu.sync_copy(idx_hbm.at[pl.ds(wid * N_CH, N_CH)], idx_v)
    bufs = (rows0, rows1)
    sems = (sem0, sem1)
    cps = [None] * N_CH
    cps[0] = pltpu.async_copy(table_hbm.at[idx_v.at[0]], bufs[0], sems[0])
    for c in range(N_CH):
        if c + 1 < N_CH:
            cps[c + 1] = pltpu.async_copy(
                table_hbm.at[idx_v.at[c + 1]], bufs[(c + 1) % 2], sems[(c + 1) % 2])
        cps[c].wait()
        pltpu.sync_copy(bufs[c % 2], out_hbm.at[pl.ds(base + c * CHUNK, CHUNK)])


@functools.cache
def _sc_gather_call():
    return pl.kernel(
        _sc_gather_body,
        mesh=plsc.VectorSubcoreMesh(
            core_axis_name="c", subcore_axis_name="s",
            num_cores=SC_NC, num_subcores=SC_NS),
        out_type=jax.ShapeDtypeStruct((N_TOK, E_DIM), jnp.float32),
        scratch_types=[
            pltpu.VMEM((N_CH, CHUNK), jnp.int32),
            pltpu.VMEM((CHUNK, E_DIM), jnp.float32),
            pltpu.VMEM((CHUNK, E_DIM), jnp.float32),
            pltpu.SemaphoreType.DMA,
            pltpu.SemaphoreType.DMA,
        ],
    )


def kernel(z, embed_weight):
    z3 = z.reshape(B_SZ, E_DIM, HW)
    zf = jnp.transpose(z, (0, 2, 3, 1)).reshape(-1, E_DIM)
    zz = jnp.sum(zf ** 2, axis=1)
    ee = jnp.sum(embed_weight ** 2, axis=1)
    idx_blocks, counts = _assign_call(
        z3, embed_weight, zz.reshape(A_GRID, 1, BM), ee.reshape(N_E, 1))
    idx2d = idx_blocks.reshape(SC_NW * N_CH, CHUNK)
    z_q = _sc_gather_call()(embed_weight, idx2d)
    loss2, perp2 = _finish_call(z_q, z.reshape(N_TOK, E_DIM), counts)
    return (z_q.reshape(z.shape), loss2[0, 0], perp2[0, 0],
            idx_blocks.reshape(B_SZ, 32, 32))
